# baseline (speedup 1.0000x reference)
import functools

import jax
import jax.numpy as jnp
from jax import lax
from jax.experimental import pallas as pl
from jax.experimental.pallas import tpu as pltpu
from jax.experimental.pallas import tpu_sc as plsc

B = 16384
D = 64
R_PACK = 256
NC, NS = 2, 16
NW = NC * NS
BPW = B // NW
C = 128
NCHUNK = BPW // C
L = 16


def _pack_relation_tables(th_relation_w, th_normal_w, ro_relation_w):
    R = th_relation_w.shape[0]

    def body(re_ref, nv_ref, rr_ref, o_ref):
        rre = rr_ref[...]
        o_ref[:, 0:D] = re_ref[...]
        o_ref[:, D:2 * D] = nv_ref[...]
        o_ref[:, 2 * D:3 * D] = rre
        o_ref[:, 3 * D:4 * D] = jnp.cos(rre) + jnp.sin(rre)

    return pl.pallas_call(
        body,
        out_shape=jax.ShapeDtypeStruct((R, R_PACK), jnp.float32),
    )(th_relation_w, th_normal_w, ro_relation_w)


def _vsqrt(x):
    xi = lax.bitcast_convert_type(x, jnp.int32)
    yi = jnp.int32(0x5F3759DF) - lax.shift_right_logical(xi, 1)
    y = lax.bitcast_convert_type(yi, jnp.float32)
    xh = x * 0.5
    y = y * (1.5 - xh * y * y)
    y = y * (1.5 - xh * y * y)
    y = y * (1.5 - xh * y * y)
    return x * y


def _sc_body(head_hbm, rel_hbm, tail_hbm, the_hbm, roe_hbm, rp_hbm, out_hbm,
             hidx, ridx, tidx, he, te, rhe, rte, rp, ob, sem):
    wid = lax.axis_index("s") * NC + lax.axis_index("c")
    base = wid * BPW
    for chunk in range(NCHUNK):
        off = base + chunk * C
        pltpu.sync_copy(head_hbm.at[pl.ds(off, C)], hidx)
        pltpu.sync_copy(rel_hbm.at[pl.ds(off, C)], ridx)
        pltpu.sync_copy(tail_hbm.at[pl.ds(off, C)], tidx)
        c1 = pltpu.async_copy(the_hbm.at[hidx], he, sem)
        c2 = pltpu.async_copy(the_hbm.at[tidx], te, sem)
        c3 = pltpu.async_copy(roe_hbm.at[hidx], rhe, sem)
        c4 = pltpu.async_copy(roe_hbm.at[tidx], rte, sem)
        c5 = pltpu.async_copy(rp_hbm.at[ridx], rp, sem)
        c1.wait(); c2.wait(); c3.wait(); c4.wait(); c5.wait()

        def group(g, carry):
            lane = lax.iota(jnp.int32, L)
            sum1 = jnp.zeros((L,), jnp.float32)
            sum2 = jnp.zeros((L,), jnp.float32)
            for i in range(L):
                it = g * L + i
                hv = [he[it, pl.ds(L * k, L)] for k in range(4)]
                tv = [te[it, pl.ds(L * k, L)] for k in range(4)]
                rev = [rp[it, pl.ds(L * k, L)] for k in range(4)]
                nvv = [rp[it, pl.ds(D + L * k, L)] for k in range(4)]
                d = [hv[k] - tv[k] for k in range(4)]
                t = d[0] * nvv[0] + d[1] * nvv[1] + d[2] * nvv[2] + d[3] * nvv[3]
                a = jnp.sum(t)
                sq1 = jnp.zeros((L,), jnp.float32)
                for k in range(4):
                    u = d[k] + rev[k] - a * nvv[k]
                    sq1 = sq1 + u * u
                s1 = jnp.sum(sq1)
                rhv = [rhe[it, pl.ds(L * k, L)] for k in range(4)]
                rtv = [rte[it, pl.ds(L * k, L)] for k in range(4)]
                rrv = [rp[it, pl.ds(2 * D + L * k, L)] for k in range(4)]
                cv = [rp[it, pl.ds(3 * D + L * k, L)] for k in range(4)]
                sq2 = jnp.zeros((L,), jnp.float32)
                for k in range(4):
                    w = rhv[k] * cv[k] + rrv[k] - rtv[k]
                    sq2 = sq2 + w * w
                s2 = jnp.sum(sq2)
                sum1 = jnp.where(lane == i, s1, sum1)
                sum2 = jnp.where(lane == i, s2, sum2)
            ob[pl.ds(g * L, L)] = _vsqrt(sum1) + _vsqrt(sum2)
            return carry

        lax.fori_loop(0, C // L, group, 0)
        pltpu.sync_copy(ob, out_hbm.at[pl.ds(off, C)])


@functools.partial(jax.jit, static_argnames=())
def kernel(head, relation, tail, th_entity_w, th_relation_w, th_normal_w,
           ro_entity_w, ro_relation_w):
    rp = _pack_relation_tables(th_relation_w, th_normal_w, ro_relation_w)
    mesh = plsc.VectorSubcoreMesh(core_axis_name="c", subcore_axis_name="s",
                                  num_cores=NC, num_subcores=NS)
    sc = pl.kernel(
        _sc_body,
        out_type=jax.ShapeDtypeStruct((B,), jnp.float32),
        mesh=mesh,
        compiler_params=pltpu.CompilerParams(needs_layout_passes=False,
                                             use_tc_tiling_on_sc=False),
        scratch_types=[
            pltpu.VMEM((C,), jnp.int32),
            pltpu.VMEM((C,), jnp.int32),
            pltpu.VMEM((C,), jnp.int32),
            pltpu.VMEM((C, D), jnp.float32),
            pltpu.VMEM((C, D), jnp.float32),
            pltpu.VMEM((C, D), jnp.float32),
            pltpu.VMEM((C, D), jnp.float32),
            pltpu.VMEM((C, R_PACK), jnp.float32),
            pltpu.VMEM((C,), jnp.float32),
            pltpu.SemaphoreType.DMA,
        ],
    )
    return sc(head, relation, tail, th_entity_w, ro_entity_w, rp)
